# baseline (speedup 1.0000x reference)
import functools

import jax
import jax.numpy as jnp
from jax import lax
from jax.experimental import pallas as pl
from jax.experimental.pallas import tpu as pltpu
from jax.experimental.pallas import tpu_sc as plsc

_DEMB = 768
_NC = 2
_NS = 16
_NW = _NC * _NS
_B = 32768
_BPW = _B // _NW
_C = 32
_NB = 4
_NCHUNK = _BPW // _C

_mesh = plsc.VectorSubcoreMesh(core_axis_name="c", subcore_axis_name="s")


@functools.partial(
    pl.kernel,
    out_type=jax.ShapeDtypeStruct((_B, _DEMB), jnp.float32),
    mesh=_mesh,
    scratch_types=[
        pltpu.VMEM((_BPW,), jnp.int32),
        pltpu.VMEM((_NB, _C, _DEMB), jnp.float32),
        pltpu.SemaphoreType.DMA((_NB,)),
        pltpu.SemaphoreType.DMA((_NB,)),
    ],
)
def _emb_gather(idx_hbm, table_hbm, out_hbm, idx_v, buf, gsem, ssem):
    wid = lax.axis_index("s") * _NC + lax.axis_index("c")
    base = wid * _BPW
    pltpu.sync_copy(idx_hbm.at[pl.ds(base, _BPW)], idx_v)

    def gather_handle(c):
        b = c % _NB
        return pltpu.make_async_copy(
            table_hbm.at[idx_v.at[pl.ds(c * _C, _C)]], buf.at[b], gsem.at[b]
        )

    def store_handle(c):
        b = c % _NB
        return pltpu.make_async_copy(
            buf.at[b], out_hbm.at[pl.ds(base + c * _C, _C)], ssem.at[b]
        )

    for c in range(_NB - 1):
        gather_handle(c).start()

    def chunk_body(c, carry):
        gather_handle(c).wait()
        store_handle(c).start()
        g = c + _NB - 1

        @pl.when(jnp.logical_and(c >= 1, g < _NCHUNK))
        def _():
            store_handle(g - _NB).wait()

        @pl.when(g < _NCHUNK)
        def _():
            gather_handle(g).start()

        return carry

    lax.fori_loop(0, _NCHUNK, chunk_body, 0)
    for c in range(_NCHUNK - _NB, _NCHUNK):
        store_handle(c).wait()


def kernel(pos_seq, pos_emb):
    d = pos_emb.shape[-1]
    idx = pos_seq.reshape(-1).astype(jnp.int32)
    out = _emb_gather(idx, pos_emb)
    return out.reshape(pos_seq.shape + (d,))


# speedup vs baseline: 1.0100x; 1.0100x over previous
import functools

import jax
import jax.numpy as jnp
from jax import lax
from jax.experimental import pallas as pl
from jax.experimental.pallas import tpu as pltpu
from jax.experimental.pallas import tpu_sc as plsc

_DEMB = 768
_NC = 2
_NS = 16
_NW = _NC * _NS
_B = 32768
_BPW = _B // _NW
_C = 16
_NB = 8
_NCHUNK = _BPW // _C

_mesh = plsc.VectorSubcoreMesh(core_axis_name="c", subcore_axis_name="s")


@functools.partial(
    pl.kernel,
    out_type=jax.ShapeDtypeStruct((_B, _DEMB), jnp.float32),
    mesh=_mesh,
    scratch_types=[
        pltpu.VMEM((_BPW,), jnp.int32),
        pltpu.VMEM((_NB, _C, _DEMB), jnp.float32),
        pltpu.SemaphoreType.DMA((_NB,)),
        pltpu.SemaphoreType.DMA((_NB,)),
    ],
)
def _emb_gather(idx_hbm, table_hbm, out_hbm, idx_v, buf, gsem, ssem):
    wid = lax.axis_index("s") * _NC + lax.axis_index("c")
    base = wid * _BPW
    pltpu.sync_copy(idx_hbm.at[pl.ds(base, _BPW)], idx_v)

    def gather_handle(c):
        b = c % _NB
        return pltpu.make_async_copy(
            table_hbm.at[idx_v.at[pl.ds(c * _C, _C)]], buf.at[b], gsem.at[b]
        )

    def store_handle(c):
        b = c % _NB
        return pltpu.make_async_copy(
            buf.at[b], out_hbm.at[pl.ds(base + c * _C, _C)], ssem.at[b]
        )

    for c in range(_NB - 1):
        gather_handle(c).start()

    def chunk_body(c, carry):
        gather_handle(c).wait()
        store_handle(c).start()
        g = c + _NB - 1

        @pl.when(jnp.logical_and(c >= 1, g < _NCHUNK))
        def _():
            store_handle(g - _NB).wait()

        @pl.when(g < _NCHUNK)
        def _():
            gather_handle(g).start()

        return carry

    lax.fori_loop(0, _NCHUNK, chunk_body, 0)
    for c in range(_NCHUNK - _NB, _NCHUNK):
        store_handle(c).wait()


def kernel(pos_seq, pos_emb):
    d = pos_emb.shape[-1]
    idx = pos_seq.reshape(-1).astype(jnp.int32)
    out = _emb_gather(idx, pos_emb)
    return out.reshape(pos_seq.shape + (d,))
